# Initial kernel scaffold; baseline (speedup 1.0000x reference)
import functools

import jax
import jax.numpy as jnp
from jax import lax
from jax.experimental import pallas as pl
from jax.experimental.pallas import tpu as pltpu
from jax.experimental.pallas import tpu_sc as plsc

N = 10000
E = 160000
C = 128
T = 4
NB = 512
NTAB = 10240
GRID = NTAB // NB
CHW = 128
NCHUNK = 1280
EPAD = NCHUNK * CHW
NROW_SC = 10016
RPT = NROW_SC // 16
CPT_A = NCHUNK // 32
CPT_C = NCHUNK // 16

_mesh = plsc.VectorSubcoreMesh(core_axis_name="c", subcore_axis_name="s")


@functools.partial(
    pl.kernel,
    out_type=jax.ShapeDtypeStruct((4, NROW_SC, 16), jnp.float32),
    mesh=_mesh,
    scratch_types=[
        pltpu.VMEM((CPT_A, CHW), jnp.int32),
        pltpu.VMEM((CPT_A, CHW), jnp.int32),
        pltpu.VMEM((CHW, 16), jnp.float32),
        pltpu.VMEM_SHARED((NROW_SC, 16), jnp.float32),
        pltpu.VMEM_SHARED((NROW_SC, 16), jnp.float32),
    ],
)
def _degree_kernel(srcp, dstp, ones_hbm, zrow, hist, src_v, dst_v, ones_v,
                   dsrc_sh, ddst_sh):
    c = lax.axis_index("c")
    s = lax.axis_index("s")
    w = c * 16 + s
    pltpu.sync_copy(srcp.at[pl.ds(w * CPT_A, CPT_A)], src_v)
    pltpu.sync_copy(dstp.at[pl.ds(w * CPT_A, CPT_A)], dst_v)
    pltpu.sync_copy(ones_hbm, ones_v)
    pltpu.sync_copy(zrow, dsrc_sh.at[pl.ds(s * RPT, RPT)])
    pltpu.sync_copy(zrow, ddst_sh.at[pl.ds(s * RPT, RPT)])
    plsc.subcore_barrier()

    def body(j, carry):
        pltpu.sync_copy(ones_v, dsrc_sh.at[src_v.at[j]], add=True)
        pltpu.sync_copy(ones_v, ddst_sh.at[dst_v.at[j]], add=True)
        return carry

    lax.fori_loop(0, CPT_A, body, 0)
    plsc.subcore_barrier()
    for ci in range(2):
        @pl.when(c == ci)
        def _(ci=ci):
            rows = pl.ds(s * RPT, RPT)
            pltpu.sync_copy(dsrc_sh.at[rows], hist.at[2 * ci + 0, rows])
            pltpu.sync_copy(ddst_sh.at[rows], hist.at[2 * ci + 1, rows])


@functools.partial(
    pl.kernel,
    out_type=[jax.ShapeDtypeStruct((NROW_SC, C), jnp.float32)] * 4,
    mesh=_mesh,
    scratch_types=[
        pltpu.VMEM((CPT_C, CHW), jnp.int32),
        pltpu.VMEM((CPT_C, CHW), jnp.int32),
        pltpu.VMEM((2, CHW, C), jnp.float32),
        pltpu.SemaphoreType.DMA,
        pltpu.SemaphoreType.DMA,
        pltpu.VMEM_SHARED((NROW_SC, C), jnp.float32),
    ],
)
def _agg_kernel(h0, h1, h2, h3, srcp, dstp, zrow, a0, a1, a2, a3,
                src_v, dst_v, rows2, sem0, sem1, agg_sh):
    c = lax.axis_index("c")
    s = lax.axis_index("s")
    pltpu.sync_copy(srcp.at[pl.ds(s * CPT_C, CPT_C)], src_v)
    pltpu.sync_copy(dstp.at[pl.ds(s * CPT_C, CPT_C)], dst_v)
    tables = ((h0, h2), (h1, h3))
    outs = ((a0, a2), (a1, a3))
    my_rows = pl.ds(s * RPT, RPT)
    for u in range(2):
        pltpu.sync_copy(zrow, agg_sh.at[my_rows])
        plsc.subcore_barrier()
        for ci in range(2):
            ht = tables[u][ci]

            @pl.when(c == ci)
            def _(ht=ht):
                pltpu.async_copy(ht.at[src_v.at[0]], rows2.at[0], sem0)

                def body(g, carry):
                    j0 = 2 * g
                    j1 = j0 + 1
                    pltpu.async_copy(ht.at[src_v.at[j1]], rows2.at[1], sem1)
                    pltpu.make_async_copy(ht.at[src_v.at[j0]], rows2.at[0],
                                          sem0).wait()
                    pltpu.sync_copy(rows2.at[0], agg_sh.at[dst_v.at[j0]],
                                    add=True)

                    @pl.when(g < CPT_C // 2 - 1)
                    def _():
                        pltpu.async_copy(ht.at[src_v.at[j0 + 2]], rows2.at[0],
                                         sem0)

                    pltpu.make_async_copy(ht.at[src_v.at[j1]], rows2.at[1],
                                          sem1).wait()
                    pltpu.sync_copy(rows2.at[1], agg_sh.at[dst_v.at[j1]],
                                    add=True)
                    return carry

                lax.fori_loop(0, CPT_C // 2, body, 0)

        plsc.subcore_barrier()
        for ci in range(2):
            aout = outs[u][ci]

            @pl.when(c == ci)
            def _(aout=aout):
                pltpu.sync_copy(agg_sh.at[my_rows], aout.at[my_rows])

        plsc.subcore_barrier()


def _pre_body(x_ref, hist_ref, w_ref, h0_ref, h1_ref, h2_ref, h3_ref):
    i = pl.program_id(0)
    nids = lax.broadcasted_iota(jnp.int32, (NB, 1), 0) + i * NB
    valid = nids < N
    hs = hist_ref[...]
    deg_out = (hs[0] + hs[2])[:, 0:1]
    norm_src = jnp.where(deg_out > 0, lax.rsqrt(deg_out), 0.0)
    norm_src = jnp.where(valid, norm_src, 0.0)
    w = w_ref[...]
    for t, href in enumerate((h0_ref, h1_ref, h2_ref, h3_ref)):
        xt = x_ref[0, :, t, :]
        h = lax.dot_general(xt, w, (((0,), (0,)), ((), ())),
                            preferred_element_type=jnp.float32)
        href[...] = h * norm_src


def _post_body(hist_ref, a0_ref, a1_ref, a2_ref, a3_ref, b_ref, o_ref):
    hs = hist_ref[...]
    deg_in = (hs[1] + hs[3])[:, 0:1]
    norm_dst = jnp.where(deg_in > 0, lax.rsqrt(deg_in), 0.0)
    r = lax.broadcasted_iota(jnp.int32, (C, C), 0)
    col = lax.broadcasted_iota(jnp.int32, (C, C), 1)
    eye = (r == col).astype(jnp.float32)
    bb = b_ref[...]
    for t, aref in enumerate((a0_ref, a1_ref, a2_ref, a3_ref)):
        a = aref[...] * norm_dst
        z = lax.dot_general(eye, a, (((0,), (1,)), ((), ())),
                            preferred_element_type=jnp.float32)
        o_ref[0, :, t, :] = jnp.maximum(z + bb, 0.0)


def kernel(x, edge_index, W, b):
    f32 = jnp.float32
    pad = jnp.full((EPAD - E,), N, jnp.int32)
    srcp = jnp.concatenate([edge_index[0], pad]).reshape(NCHUNK, CHW)
    dstp = jnp.concatenate([edge_index[1], pad]).reshape(NCHUNK, CHW)
    ones_rows = jnp.ones((CHW, 16), f32)
    zrow16 = jnp.zeros((RPT, 16), f32)
    zrowC = jnp.zeros((RPT, C), f32)

    hist = _degree_kernel(srcp, dstp, ones_rows, zrow16)

    tabs = pl.pallas_call(
        _pre_body,
        grid=(GRID,),
        in_specs=[
            pl.BlockSpec((1, C, T, NB), lambda i: (0, 0, 0, i)),
            pl.BlockSpec((4, NB, 16), lambda i: (0, i, 0)),
            pl.BlockSpec((C, C), lambda i: (0, 0)),
        ],
        out_specs=[pl.BlockSpec((NB, C), lambda i: (i, 0))] * 4,
        out_shape=[jax.ShapeDtypeStruct((NTAB, C), f32)] * 4,
    )(x, hist, W)

    aggs = _agg_kernel(tabs[0], tabs[1], tabs[2], tabs[3], srcp, dstp, zrowC)

    out = pl.pallas_call(
        _post_body,
        grid=(GRID,),
        in_specs=[pl.BlockSpec((4, NB, 16), lambda i: (0, i, 0))]
        + [pl.BlockSpec((NB, C), lambda i: (i, 0))] * 4
        + [pl.BlockSpec((C, 1), lambda i: (0, 0))],
        out_specs=pl.BlockSpec((1, C, T, NB), lambda i: (0, 0, 0, i)),
        out_shape=jax.ShapeDtypeStruct((1, C, T, N), f32),
    )(hist, aggs[0], aggs[1], aggs[2], aggs[3], b.reshape(C, 1))
    return out


# speedup vs baseline: 42.3842x; 42.3842x over previous
import functools

import jax
import jax.numpy as jnp
from jax import lax
from jax.experimental import pallas as pl
from jax.experimental.pallas import tpu as pltpu
from jax.experimental.pallas import tpu_sc as plsc

N = 10000
E = 160000
C = 128
T = 4
NB = 512
NTAB = 10240
GRID = NTAB // NB
NROW_H = 10112
EPT_A = E // 32
EPT_AP = 5008
EPT_C = E // 16
EPT_CP = 10048
CW = 64
NCHT = EPT_CP // CW
ZPT = N // 16

_mesh = plsc.VectorSubcoreMesh(core_axis_name="c", subcore_axis_name="s")
_sc_params = pltpu.CompilerParams(use_tc_tiling_on_sc=False,
                                  needs_layout_passes=False)


@functools.partial(
    pl.kernel,
    out_type=jax.ShapeDtypeStruct((64, NROW_H), jnp.int32),
    mesh=_mesh,
    compiler_params=_sc_params,
    scratch_types=[
        pltpu.VMEM((EPT_AP,), jnp.int32),
        pltpu.VMEM((EPT_AP,), jnp.int32),
        pltpu.VMEM((NROW_H,), jnp.int32),
        pltpu.VMEM((NROW_H,), jnp.int32),
    ],
)
def _degree_kernel(eflat, hist, sbuf, dbuf, hsrc, hdst):
    c = lax.axis_index("c")
    s = lax.axis_index("s")
    w = c * 16 + s
    pltpu.sync_copy(eflat.at[pl.ds(w * EPT_A, EPT_A)],
                    sbuf.at[pl.ds(0, EPT_A)])
    pltpu.sync_copy(eflat.at[pl.ds(E + w * EPT_A, EPT_A)],
                    dbuf.at[pl.ds(0, EPT_A)])
    lanes = lax.broadcasted_iota(jnp.int32, (16,), 0)
    tmask = lanes < (16 - (EPT_AP - EPT_A))
    tail = pl.ds(EPT_AP - 16, 16)
    sbuf[tail] = jnp.where(tmask, sbuf[tail], N)
    dbuf[tail] = jnp.where(tmask, dbuf[tail], N)
    zero16 = jnp.zeros((16,), jnp.int32)

    def zfill(i, carry):
        hsrc[pl.ds(i * 16, 16)] = zero16
        hdst[pl.ds(i * 16, 16)] = zero16
        return carry

    lax.fori_loop(0, NROW_H // 16, zfill, 0)

    def body(i, carry):
        v = sbuf[pl.ds(i * 16, 16)]
        cnt, lm = plsc.scan_count(v)
        plsc.addupdate_scatter(hsrc, [v], cnt, mask=lm)
        u = dbuf[pl.ds(i * 16, 16)]
        cnt2, lm2 = plsc.scan_count(u)
        plsc.addupdate_scatter(hdst, [u], cnt2, mask=lm2)
        return carry

    lax.fori_loop(0, EPT_AP // 16, body, 0)
    pltpu.sync_copy(hsrc, hist.at[w])
    pltpu.sync_copy(hdst, hist.at[32 + w])


@functools.partial(
    pl.kernel,
    out_type=jax.ShapeDtypeStruct((4, N, C), jnp.float32),
    mesh=_mesh,
    compiler_params=_sc_params,
    scratch_types=[
        pltpu.VMEM((EPT_CP,), jnp.int32),
        pltpu.VMEM((EPT_CP,), jnp.int32),
        pltpu.VMEM((2, CW), jnp.int32),
        pltpu.VMEM((2, CW), jnp.int32),
        pltpu.VMEM((CW, C), jnp.float32),
        pltpu.VMEM((2, CW, C), jnp.float32),
        pltpu.SemaphoreType.DMA,
        pltpu.SemaphoreType.DMA,
        pltpu.VMEM_SHARED((N, C), jnp.float32),
    ],
)
def _agg_kernel(hflat, eflat, agg_out,
                se_v, de_v, adjr, dstr, zbuf, rows2, sem0, sem1, agg_sh):
    c = lax.axis_index("c")
    s = lax.axis_index("s")
    pltpu.sync_copy(eflat.at[pl.ds(s * EPT_C, EPT_C)],
                    se_v.at[pl.ds(0, EPT_C)])
    pltpu.sync_copy(eflat.at[pl.ds(E + s * EPT_C, EPT_C)],
                    de_v.at[pl.ds(0, EPT_C)])
    fullN = jnp.full((16,), N, jnp.int32)
    zero16i = jnp.zeros((16,), jnp.int32)
    for i in range(EPT_C, EPT_CP, 16):
        se_v[pl.ds(i, 16)] = fullN
        de_v[pl.ds(i, 16)] = zero16i
    zero16 = jnp.zeros((16,), jnp.float32)

    def zfill(r, carry):
        def zinner(k, carry2):
            zbuf[r, pl.ds(k * 16, 16)] = zero16
            return carry2
        return lax.fori_loop(0, C // 16, zinner, carry)

    lax.fori_loop(0, CW, zfill, 0)

    def build_adj(slot, j, off):
        for k in range(CW // 16):
            adjr[slot, pl.ds(k * 16, 16)] = (
                se_v[pl.ds(j * CW + k * 16, 16)] + off)

    def build_dst(slot, j):
        for k in range(CW // 16):
            dstr[slot, pl.ds(k * 16, 16)] = de_v[pl.ds(j * CW + k * 16, 16)]

    for u in range(2):
        toff = (2 * c + u) * NTAB
        for q in range(ZPT // CW + 1):
            nr = CW if q < ZPT // CW else ZPT % CW
            pltpu.sync_copy(zbuf.at[pl.ds(0, nr)],
                            agg_sh.at[pl.ds(s * ZPT + q * CW, nr)])
        plsc.subcore_barrier()

        build_adj(0, 0, toff)
        pltpu.async_copy(hflat.at[adjr.at[0]], rows2.at[0], sem0)

        def body(g, carry):
            j0 = 2 * g
            j1 = j0 + 1
            build_adj(1, j1, toff)
            pltpu.async_copy(hflat.at[adjr.at[1]], rows2.at[1], sem1)
            pltpu.make_async_copy(hflat.at[adjr.at[0]], rows2.at[0],
                                  sem0).wait()
            build_dst(0, j0)
            pltpu.sync_copy(rows2.at[0], agg_sh.at[dstr.at[0]], add=True)
            build_adj(0, j0 + 2, toff)
            pltpu.async_copy(hflat.at[adjr.at[0]], rows2.at[0], sem0)
            pltpu.make_async_copy(hflat.at[adjr.at[1]], rows2.at[1],
                                  sem1).wait()
            build_dst(1, j1)
            pltpu.sync_copy(rows2.at[1], agg_sh.at[dstr.at[1]], add=True)
            return carry

        lax.fori_loop(0, (NCHT - 1) // 2, body, 0)
        pltpu.make_async_copy(hflat.at[adjr.at[0]], rows2.at[0], sem0).wait()
        build_dst(0, NCHT - 1)
        pltpu.sync_copy(rows2.at[0], agg_sh.at[dstr.at[0]], add=True)
        plsc.subcore_barrier()
        for ci in range(2):
            @pl.when(c == ci)
            def _(ci=ci, u=u):
                rows = pl.ds(s * ZPT, ZPT)
                pltpu.sync_copy(agg_sh.at[rows], agg_out.at[2 * ci + u, rows])

        plsc.subcore_barrier()


def _pre_body(x_ref, hist_ref, w_ref, h_ref):
    i = pl.program_id(0)
    nids = lax.broadcasted_iota(jnp.int32, (NB, 1), 0) + i * NB
    valid = nids < N
    hs = hist_ref[...].astype(jnp.float32)
    deg_out = jnp.sum(hs[0:32], axis=0, keepdims=True)
    norm_row = jnp.where(deg_out > 0, lax.rsqrt(deg_out), 0.0)
    norm_src = jnp.transpose(norm_row)
    w = w_ref[...]
    for t in range(T):
        xt = x_ref[0, :, t, :]
        h = lax.dot_general(xt, w, (((0,), (0,)), ((), ())),
                            preferred_element_type=jnp.float32)
        h_ref[t] = jnp.where(valid, h * norm_src, 0.0)


def _post_body(hist_ref, a_ref, b_ref, o_ref):
    hs = hist_ref[...].astype(jnp.float32)
    deg_in = jnp.sum(hs[32:64], axis=0, keepdims=True)
    norm_row = jnp.where(deg_in > 0, lax.rsqrt(deg_in), 0.0)
    norm_dst = jnp.transpose(norm_row)
    r = lax.broadcasted_iota(jnp.int32, (C, C), 0)
    col = lax.broadcasted_iota(jnp.int32, (C, C), 1)
    eye = (r == col).astype(jnp.float32)
    bb = b_ref[...]
    for t in range(T):
        a = a_ref[t] * norm_dst
        z = lax.dot_general(eye, a, (((0,), (1,)), ((), ())),
                            preferred_element_type=jnp.float32)
        o_ref[0, :, t, :] = jnp.maximum(z + bb, 0.0)


def kernel(x, edge_index, W, b):
    f32 = jnp.float32
    eflat = edge_index.reshape(2 * E)

    hist = _degree_kernel(eflat)

    tab = pl.pallas_call(
        _pre_body,
        grid=(GRID,),
        in_specs=[
            pl.BlockSpec((1, C, T, NB), lambda i: (0, 0, 0, i)),
            pl.BlockSpec((64, NB), lambda i: (0, i)),
            pl.BlockSpec((C, C), lambda i: (0, 0)),
        ],
        out_specs=pl.BlockSpec((T, NB, C), lambda i: (0, i, 0)),
        out_shape=jax.ShapeDtypeStruct((T, NTAB, C), f32),
    )(x, hist, W)

    agg = _agg_kernel(tab.reshape(T * NTAB, C), eflat)

    out = pl.pallas_call(
        _post_body,
        grid=(GRID,),
        in_specs=[
            pl.BlockSpec((64, NB), lambda i: (0, i)),
            pl.BlockSpec((T, NB, C), lambda i: (0, i, 0)),
            pl.BlockSpec((C, 1), lambda i: (0, 0)),
        ],
        out_specs=pl.BlockSpec((1, C, T, NB), lambda i: (0, 0, 0, i)),
        out_shape=jax.ShapeDtypeStruct((1, C, T, N), f32),
    )(hist, agg, b.reshape(C, 1))
    return out
